# baseline (speedup 1.0000x reference)
import functools

import jax
import jax.numpy as jnp
from jax import lax
from jax.experimental import pallas as pl
from jax.experimental.pallas import tpu as pltpu
from jax.experimental.pallas import tpu_sc as plsc

_LANES = 16
_LOG2E = 1.4426950408889634


def _sc_affine(x2, et2, mul_flat, bias_flat, row_start):
    R, C = x2.shape
    info = plsc.get_sparse_core_info()
    nw = info.num_cores * info.num_subcores
    rows = (R - row_start) // nw
    chunk = rows * C
    count = (R - row_start) * C
    assert rows * nw == R - row_start and chunk % _LANES == 0 and chunk % 8 == 0
    table = mul_flat.shape[0]
    mesh = plsc.VectorSubcoreMesh(core_axis_name="c", subcore_axis_name="s")

    @functools.partial(
        pl.kernel,
        mesh=mesh,
        out_type=jax.ShapeDtypeStruct((count,), jnp.float32),
        compiler_params=pltpu.CompilerParams(needs_layout_passes=False),
        scratch_types=[
            pltpu.VMEM((rows, C), jnp.int32),
            pltpu.VMEM((rows, C), jnp.float32),
            pltpu.VMEM((chunk,), jnp.float32),
            pltpu.VMEM((table,), jnp.float32),
            pltpu.VMEM((table,), jnp.float32),
        ],
    )
    def sc_run(x_hbm, et_hbm, mul_hbm, bias_hbm, out_hbm,
               idx_v, x_v, out_v, mul_v, bias_v):
        wid = lax.axis_index("s") * info.num_cores + lax.axis_index("c")
        base_r = row_start + wid * rows
        pltpu.sync_copy(et_hbm.at[pl.ds(base_r, rows)], idx_v)
        pltpu.sync_copy(x_hbm.at[pl.ds(base_r, rows)], x_v)
        pltpu.sync_copy(mul_hbm, mul_v)
        pltpu.sync_copy(bias_hbm, bias_v)
        groups_per_row = C // _LANES

        @plsc.parallel_loop(0, chunk // _LANES, 1, unroll=8)
        def _(i):
            r = i // groups_per_row
            sl = pl.ds((i % groups_per_row) * _LANES, _LANES)
            idx = idx_v[r, sl]
            m = plsc.load_gather(mul_v, [idx])
            b = plsc.load_gather(bias_v, [idx])
            out_v[pl.ds(i * _LANES, _LANES)] = m * x_v[r, sl] + b

        pltpu.sync_copy(out_v, out_hbm.at[pl.ds(wid * chunk, chunk)])

    return sc_run(x2, et2, mul_flat, bias_flat)


def _tc_rbf_gather_head(x2, et2, means, temps, mul_t, bias_t, nrows, P, bp):
    R, Q = x2.shape
    K = means.shape[1]

    def body(x_ref, et_ref, mean_ref, temp_ref, mul_ref, bias_ref, out_ref):
        xb = x_ref[...]
        et = et_ref[...]
        m = mean_ref[...][0]
        c = jnp.abs(temp_ref[...][0]) * (-_LOG2E)
        low = et & 127
        hi = et >> 7
        mul_v = jnp.zeros_like(xb)
        bias_v = jnp.zeros_like(xb)
        for ch in range(8):
            mrow = mul_ref[...][ch]
            brow = bias_ref[...][ch]
            mg = jnp.take_along_axis(
                jnp.broadcast_to(mrow[None, :], (xb.shape[0], 128)), low, axis=1)
            bg = jnp.take_along_axis(
                jnp.broadcast_to(brow[None, :], (xb.shape[0], 128)), low, axis=1)
            sel = hi == ch
            mul_v = jnp.where(sel, mg, mul_v)
            bias_v = jnp.where(sel, bg, bias_v)
        xs = mul_v * xb + bias_v
        d = xs[:, :, None] - m[None, None, :]
        out_ref[...] = jnp.exp2(d * d * c[None, None, :]).reshape(out_ref.shape)

    ratio = Q // 128
    return pl.pallas_call(
        body,
        grid=(nrows // bp,),
        in_specs=[
            pl.BlockSpec((bp, Q), lambda i: (i, 0)),
            pl.BlockSpec((bp, Q), lambda i: (i, 0)),
            pl.BlockSpec((8, K), lambda i: (0, 0)),
            pl.BlockSpec((8, K), lambda i: (0, 0)),
            pl.BlockSpec((8, 128), lambda i: (0, 0)),
            pl.BlockSpec((8, 128), lambda i: (0, 0)),
        ],
        out_specs=pl.BlockSpec((bp * ratio, 128, K), lambda i: (i, 0, 0)),
        out_shape=jax.ShapeDtypeStruct((P, 128, K), jnp.float32),
    )(x2, et2, means, temps, mul_t, bias_t)


def _tc_rbf_tail(xs2, means, temps, out_partial, bp):
    Pb, Q = xs2.shape
    K = means.shape[1]
    P = out_partial.shape[0]
    off = (P - Pb) // bp

    def body(xs_ref, mean_ref, temp_ref, dummy_ref, out_ref):
        xsb = xs_ref[...]
        m = mean_ref[...][0]
        c = jnp.abs(temp_ref[...][0]) * (-_LOG2E)
        d = xsb[:, :, None] - m[None, None, :]
        out_ref[...] = jnp.exp2(d * d * c[None, None, :])

    return pl.pallas_call(
        body,
        grid=(Pb // bp,),
        in_specs=[
            pl.BlockSpec((bp, Q), lambda i: (i, 0)),
            pl.BlockSpec((8, K), lambda i: (0, 0)),
            pl.BlockSpec((8, K), lambda i: (0, 0)),
            pl.BlockSpec(memory_space=pl.ANY),
        ],
        out_specs=pl.BlockSpec((bp, 128, K), lambda i: (off + i, 0, 0)),
        out_shape=jax.ShapeDtypeStruct(out_partial.shape, jnp.float32),
        input_output_aliases={3: 0},
    )(xs2, means, temps, out_partial)


def kernel(x, edge_types, t, means, temps, mul_w, bias_w):
    B, N, _ = x.shape
    K = means.shape[1]
    total = B * N * N
    P = total // 128
    x2 = x.reshape(B * N, N)
    et2 = edge_types.reshape(B * N, N).astype(jnp.int32)
    mul_t = mul_w.reshape(8, 128)
    bias_t = bias_w.reshape(8, 128)
    head_rows = (B * N) // 2
    xs_b = _sc_affine(x2, et2, mul_w.reshape(-1), bias_w.reshape(-1),
                      row_start=head_rows)
    out1 = _tc_rbf_gather_head(x2, et2, means, temps, mul_t, bias_t,
                               head_rows, P, bp=32)
    out = _tc_rbf_tail(xs_b.reshape(xs_b.shape[0] // 128, 128), means, temps,
                       out1, bp=128)
    return out.reshape(B, N, N, K)


# speedup vs baseline: 1.0268x; 1.0268x over previous
import functools

import jax
import jax.numpy as jnp
from jax import lax
from jax.experimental import pallas as pl
from jax.experimental.pallas import tpu as pltpu
from jax.experimental.pallas import tpu_sc as plsc

_LANES = 16
_LOG2E = 1.4426950408889634


def _sc_affine(x2, et2, mul_flat, bias_flat):
    R, C = x2.shape
    total = R * C
    info = plsc.get_sparse_core_info()
    nw = info.num_cores * info.num_subcores
    rows = R // nw
    chunk = rows * C
    assert rows * nw == R and chunk % _LANES == 0 and chunk % 8 == 0
    table = mul_flat.shape[0]
    mesh = plsc.VectorSubcoreMesh(core_axis_name="c", subcore_axis_name="s")

    @functools.partial(
        pl.kernel,
        mesh=mesh,
        out_type=jax.ShapeDtypeStruct((total,), jnp.float32),
        compiler_params=pltpu.CompilerParams(needs_layout_passes=False),
        scratch_types=[
            pltpu.VMEM((rows, C), jnp.int32),
            pltpu.VMEM((rows, C), jnp.float32),
            pltpu.VMEM((chunk,), jnp.float32),
            pltpu.VMEM((table,), jnp.float32),
            pltpu.VMEM((table,), jnp.float32),
        ],
    )
    def sc_run(x_hbm, et_hbm, mul_hbm, bias_hbm, out_hbm,
               idx_v, x_v, out_v, mul_v, bias_v):
        wid = lax.axis_index("s") * info.num_cores + lax.axis_index("c")
        base_r = wid * rows
        pltpu.sync_copy(et_hbm.at[pl.ds(base_r, rows)], idx_v)
        pltpu.sync_copy(x_hbm.at[pl.ds(base_r, rows)], x_v)
        pltpu.sync_copy(mul_hbm, mul_v)
        pltpu.sync_copy(bias_hbm, bias_v)
        groups_per_row = C // _LANES

        @plsc.parallel_loop(0, chunk // _LANES, 1, unroll=16)
        def _(i):
            r = i // groups_per_row
            sl = pl.ds((i % groups_per_row) * _LANES, _LANES)
            idx = idx_v[r, sl]
            m = plsc.load_gather(mul_v, [idx])
            b = plsc.load_gather(bias_v, [idx])
            out_v[pl.ds(i * _LANES, _LANES)] = m * x_v[r, sl] + b

        pltpu.sync_copy(out_v, out_hbm.at[pl.ds(wid * chunk, chunk)])

    return sc_run(x2, et2, mul_flat, bias_flat)


def _tc_rbf(xs2, means, temps, bp):
    P, Q = xs2.shape
    K = means.shape[1]

    def body(xs_ref, mean_ref, temp_ref, out_ref):
        xsb = xs_ref[...]
        m = mean_ref[...][0]
        c = jnp.abs(temp_ref[...][0]) * (-_LOG2E)
        d = xsb[:, :, None] - m[None, None, :]
        out_ref[...] = jnp.exp2(d * d * c[None, None, :])

    return pl.pallas_call(
        body,
        grid=(P // bp,),
        in_specs=[
            pl.BlockSpec((bp, Q), lambda i: (i, 0)),
            pl.BlockSpec((8, K), lambda i: (0, 0)),
            pl.BlockSpec((8, K), lambda i: (0, 0)),
        ],
        out_specs=pl.BlockSpec((bp, Q, K), lambda i: (i, 0, 0)),
        out_shape=jax.ShapeDtypeStruct((P, Q, K), jnp.float32),
    )(xs2, means, temps)


def kernel(x, edge_types, t, means, temps, mul_w, bias_w):
    B, N, _ = x.shape
    K = means.shape[1]
    total = B * N * N
    x2 = x.reshape(B * N, N)
    et2 = edge_types.reshape(B * N, N).astype(jnp.int32)
    xs = _sc_affine(x2, et2, mul_w.reshape(-1), bias_w.reshape(-1))
    out = _tc_rbf(xs.reshape(total // 128, 128), means, temps, bp=256)
    return out.reshape(B, N, N, K)
